# baseline (speedup 1.0000x reference)
import functools

import jax
import jax.numpy as jnp
from jax.experimental import pallas as pl

HID = 52
N = 4096
E = 16384
B = 256
EB_EW = 512
EB_MP = 512
RB_IM = 512

_f32 = jnp.float32


def _dot(a, b):
    return jnp.dot(a, b, preferred_element_type=_f32,
                   precision=jax.lax.Precision.HIGHEST)


def _ew_kernel(ea_ref, w1_ref, b1_ref, w2r_ref, b2r_ref, out_ref):
    h = jnp.maximum(_dot(ea_ref[...], w1_ref[...]) + b1_ref[...], 0.0)

    def body(i, _):
        w = w2r_ref[pl.ds(i, 1)][0]
        bb = b2r_ref[pl.ds(i, 1), :]
        out_ref[pl.ds(i, 1)] = (_dot(h, w) + bb)[None]
        return 0

    jax.lax.fori_loop(0, HID, body, 0)


def _ew_call(ea, gp):
    w2r = gp['en2_W'].reshape(HID, HID, HID).transpose(1, 0, 2)
    b2r = gp['en2_b'].reshape(HID, HID)
    grid = E // EB_EW
    return pl.pallas_call(
        _ew_kernel,
        grid=(grid,),
        in_specs=[
            pl.BlockSpec((EB_EW, 10), lambda i: (i, 0)),
            pl.BlockSpec((10, HID), lambda i: (0, 0)),
            pl.BlockSpec((1, HID), lambda i: (0, 0)),
            pl.BlockSpec((HID, HID, HID), lambda i: (0, 0, 0)),
            pl.BlockSpec((HID, HID), lambda i: (0, 0)),
        ],
        out_specs=pl.BlockSpec((HID, EB_EW, HID), lambda i: (0, i, 0)),
        out_shape=jax.ShapeDtypeStruct((HID, E, HID), _f32),
    )(ea, gp['en1_W'], gp['en1_b'].reshape(1, HID), w2r, b2r)


def _lin0_kernel(x_ref, w_ref, b_ref, out_ref):
    out_ref[...] = jnp.maximum(_dot(x_ref[...], w_ref[...]) + b_ref[...], 0.0)


def _lin0_call(x, gp):
    return pl.pallas_call(
        _lin0_kernel,
        out_shape=jax.ShapeDtypeStruct((N, HID), _f32),
    )(x, gp['lin0_W'], gp['lin0_b'].reshape(1, HID))


def _mp_kernel(x_ref, ew_ref, src_ref, dst_ref, out_ref):
    lane_n = jax.lax.broadcasted_iota(jnp.int32, (EB_MP, N), 1).astype(_f32)
    oh_src = (src_ref[...] == lane_n).astype(_f32)
    x_src = _dot(oh_src, x_ref[...])

    msg = jnp.zeros((EB_MP, HID), _f32)
    for i in range(HID):
        msg = msg + x_src[:, i:i + 1] * ew_ref[i]

    sub_n = jax.lax.broadcasted_iota(jnp.int32, (N, EB_MP), 0).astype(_f32)
    oh_dst_t = (dst_ref[0] == sub_n).astype(_f32)
    msg_aug = jnp.concatenate([msg, jnp.ones((EB_MP, 1), _f32)], axis=1)

    @pl.when(pl.program_id(0) == 0)
    def _():
        out_ref[...] = jnp.zeros_like(out_ref)

    out_ref[...] += _dot(oh_dst_t, msg_aug)


def _mp_call(x, ew, src_col, dst_row3):
    grid = E // EB_MP
    return pl.pallas_call(
        _mp_kernel,
        grid=(grid,),
        in_specs=[
            pl.BlockSpec((N, HID), lambda i: (0, 0)),
            pl.BlockSpec((HID, EB_MP, HID), lambda i: (0, i, 0)),
            pl.BlockSpec((EB_MP, 1), lambda i: (i, 0)),
            pl.BlockSpec((1, 1, EB_MP), lambda i: (i, 0, 0)),
        ],
        out_specs=pl.BlockSpec((N, HID + 1), lambda i: (0, 0)),
        out_shape=jax.ShapeDtypeStruct((N, HID + 1), _f32),
    )(x, ew, src_col, dst_row3)


def _upd_kernel(agg_ref, x_ref, rw_ref, cb_ref, mw_ref, mb_ref, out_ref):
    agg = agg_ref[...]
    cnt = jnp.maximum(agg[:, HID:HID + 1], 1.0)
    conv = agg[:, :HID] / cnt + _dot(x_ref[...], rw_ref[...]) + cb_ref[...]
    m = jnp.maximum(conv, 0.0)
    cat = jnp.concatenate([m, x_ref[...]], axis=1)
    out_ref[...] = _dot(cat, mw_ref[...]) + mb_ref[...]


def _upd_call(agg, x, gp):
    return pl.pallas_call(
        _upd_kernel,
        out_shape=jax.ShapeDtypeStruct((N, HID), _f32),
    )(agg, x, gp['root_W'], gp['conv_b'].reshape(1, HID),
      gp['msg_W'], gp['msg_b'].reshape(1, HID))


def _norm_kernel(out3_ref, init_ref, out_ref):
    uf = out3_ref[...] + init_ref[...]
    nrm = jnp.sqrt(jnp.sum(uf * uf, axis=1, keepdims=True))
    out_ref[...] = uf / (nrm + 1e-12)


def _norm_call(out3, init):
    return pl.pallas_call(
        _norm_kernel,
        out_shape=jax.ShapeDtypeStruct((N, HID), _f32),
    )(out3, init)


def _imap_kernel(nu_b_ref, nv_ref, sub_col_ref, svb_row_ref, svb_col_ref,
                 sub_row_ref, up_ref, vp_ref):
    nu_b = nu_b_ref[...]
    nv = nv_ref[...]
    mask = (sub_col_ref[...] == svb_row_ref[...]).astype(_f32)
    imap = jax.lax.dot_general(nu_b, nv, (((1,), (1,)), ((), ())),
                               preferred_element_type=_f32,
                               precision=jax.lax.Precision.HIGHEST) * mask
    up_ref[...] = _dot(imap, nv)

    mask_t = (svb_col_ref[...] == sub_row_ref[0]).astype(_f32)
    imap_t = jax.lax.dot_general(nv, nu_b, (((1,), (1,)), ((), ())),
                                 preferred_element_type=_f32,
                                 precision=jax.lax.Precision.HIGHEST) * mask_t

    @pl.when(pl.program_id(0) == 0)
    def _():
        vp_ref[...] = jnp.zeros_like(vp_ref)

    vp_ref[...] += _dot(imap_t, nu_b)


def _imap_call(nu, nv, sub_col, svb_row, svb_col, sub_row3):
    grid = N // RB_IM
    return pl.pallas_call(
        _imap_kernel,
        grid=(grid,),
        in_specs=[
            pl.BlockSpec((RB_IM, HID), lambda i: (i, 0)),
            pl.BlockSpec((N, HID), lambda i: (0, 0)),
            pl.BlockSpec((RB_IM, 1), lambda i: (i, 0)),
            pl.BlockSpec((1, N), lambda i: (0, 0)),
            pl.BlockSpec((N, 1), lambda i: (0, 0)),
            pl.BlockSpec((1, 1, RB_IM), lambda i: (i, 0, 0)),
        ],
        out_specs=[
            pl.BlockSpec((RB_IM, HID), lambda i: (i, 0)),
            pl.BlockSpec((N, HID), lambda i: (0, 0)),
        ],
        out_shape=[
            jax.ShapeDtypeStruct((N, HID), _f32),
            jax.ShapeDtypeStruct((N, HID), _f32),
        ],
    )(nu, nv, sub_col, svb_row, svb_col, sub_row3)


def _s2s_kernel(na_ref, nb_ref, b_col_ref, b_row_ref, wih_ref, whh_ref,
                bb_ref, out_ref):
    d = 2 * HID
    x = jnp.concatenate([na_ref[...], nb_ref[...]], axis=1)
    oh_nb = (b_col_ref[...] ==
             jax.lax.broadcasted_iota(jnp.int32, (N, B), 1).astype(_f32)
             ).astype(_f32)
    oh_bn = (b_row_ref[...] ==
             jax.lax.broadcasted_iota(jnp.int32, (B, N), 0).astype(_f32)
             ).astype(_f32)

    q_star = jnp.zeros((B, 2 * d), _f32)
    h = jnp.zeros((B, d), _f32)
    c = jnp.zeros((B, d), _f32)
    for _ in range(2):
        g = _dot(q_star, wih_ref[...]) + _dot(h, whh_ref[...]) + bb_ref[...]
        i_ = jax.nn.sigmoid(g[:, :d])
        f_ = jax.nn.sigmoid(g[:, d:2 * d])
        gg = jnp.tanh(g[:, 2 * d:3 * d])
        o_ = jax.nn.sigmoid(g[:, 3 * d:])
        c = f_ * c + i_ * gg
        h = o_ * jnp.tanh(c)
        q = h
        qn = _dot(oh_nb, q)
        e = jnp.sum(x * qn, axis=1, keepdims=True)
        m2 = jnp.where(oh_nb > 0.5, e, -1e30)
        emax = jnp.max(m2, axis=0, keepdims=True)
        emax = jnp.where(emax > -1e29, emax, 0.0)
        emax_n = jnp.sum(oh_nb * emax, axis=1, keepdims=True)
        ex = jnp.exp(e - emax_n)
        den = jnp.sum(jnp.where(oh_nb > 0.5, ex, 0.0), axis=0, keepdims=True)
        den_n = jnp.sum(oh_nb * den, axis=1, keepdims=True)
        a = ex / (den_n + 1e-16)
        r = _dot(oh_bn, a * x)
        q_star = jnp.concatenate([q, r], axis=1)
    out_ref[...] = q_star


def _s2s_call(na, nb, b_col, b_row, sp):
    d = 2 * HID
    return pl.pallas_call(
        _s2s_kernel,
        out_shape=jax.ShapeDtypeStruct((B, 2 * d), _f32),
    )(na, nb, b_col, b_row, sp['Wih'].T, sp['Whh'].T,
      (sp['bih'] + sp['bhh']).reshape(1, 4 * d))


def _pred_kernel(us_ref, vs_ref, w1_ref, b1_ref, w2_ref, b2_ref, w3_ref,
                 b3_ref, out_ref):
    sol = jnp.concatenate([us_ref[...], vs_ref[...]], axis=1)
    h1 = jnp.maximum(_dot(sol, w1_ref[...]) + b1_ref[...], 0.0)
    h2 = jnp.maximum(_dot(h1, w2_ref[...]) + b2_ref[...], 0.0)
    out_ref[...] = _dot(h2, w3_ref[...]) + b3_ref[...]


def _pred_call(us, vs, pp):
    return pl.pallas_call(
        _pred_kernel,
        out_shape=jax.ShapeDtypeStruct((B, 1), _f32),
    )(us, vs, pp['W1'], pp['b1'].reshape(1, -1), pp['W2'],
      pp['b2'].reshape(1, -1), pp['W3'], pp['b3'].reshape(1, 1))


def _gather_model(x, ei, ea, gp):
    ew = _ew_call(ea, gp)
    src_col = ei[0].astype(_f32).reshape(E, 1)
    dst_row3 = ei[1].astype(_f32).reshape(E // EB_MP, 1, EB_MP)
    out = _lin0_call(x, gp)
    for _ in range(3):
        agg = _mp_call(out, ew, src_col, dst_row3)
        out = _upd_call(agg, out, gp)
    return _norm_call(out, x)


def kernel(solute_x, solvent_x, solute_edge_attr, solvent_edge_attr, params,
           solute_edge_index, solvent_edge_index, solute_batch, solvent_batch):
    nu = _gather_model(solute_x, solute_edge_index, solute_edge_attr,
                       params['su'])
    nv = _gather_model(solvent_x, solvent_edge_index, solvent_edge_attr,
                       params['sv'])

    sub_f = solute_batch.astype(_f32)
    svb_f = solvent_batch.astype(_f32)
    u_prime, v_prime = _imap_call(
        nu, nv,
        sub_f.reshape(N, 1), svb_f.reshape(1, N), svb_f.reshape(N, 1),
        sub_f.reshape(N // RB_IM, 1, RB_IM))

    us = _s2s_call(nu, u_prime, sub_f.reshape(N, 1), sub_f.reshape(1, N),
                   params['s2s_u'])
    vs = _s2s_call(nv, v_prime, svb_f.reshape(N, 1), svb_f.reshape(1, N),
                   params['s2s_v'])
    return _pred_call(us, vs, params['pred'])


# speedup vs baseline: 2.7771x; 2.7771x over previous
import functools

import jax
import jax.numpy as jnp
from jax.experimental import pallas as pl

HID = 52
N = 4096
E = 16384
B = 256
EB_EW = 512
EB_MP = 512
RB_IM = 512

_f32 = jnp.float32
_HIGHEST = jax.lax.Precision.HIGHEST


def _dot(a, b):
    return jnp.dot(a, b, preferred_element_type=_f32, precision=_HIGHEST)


def _ew_kernel(eat_ref, w1_ref, b1_ref, w2_ref, b2_ref, out_ref):
    r = jnp.maximum(_dot(w1_ref[...], eat_ref[...]) + b1_ref[...], 0.0)
    ew_flat = _dot(w2_ref[...], r) + b2_ref[...]
    out_ref[...] = ew_flat.reshape(HID, HID, EB_EW)


def _ew_call(eat, gp):
    grid = E // EB_EW
    return pl.pallas_call(
        _ew_kernel,
        grid=(grid,),
        in_specs=[
            pl.BlockSpec((10, EB_EW), lambda i: (0, i)),
            pl.BlockSpec((HID, 10), lambda i: (0, 0)),
            pl.BlockSpec((HID, 1), lambda i: (0, 0)),
            pl.BlockSpec((HID * HID, HID), lambda i: (0, 0)),
            pl.BlockSpec((HID * HID, 1), lambda i: (0, 0)),
        ],
        out_specs=pl.BlockSpec((HID, HID, EB_EW), lambda i: (0, 0, i)),
        out_shape=jax.ShapeDtypeStruct((HID, HID, E), _f32),
    )(eat, gp['en1_W'].T, gp['en1_b'].reshape(HID, 1), gp['en2_W'].T,
      gp['en2_b'].reshape(HID * HID, 1))


def _lin0_kernel(xt_ref, w_ref, b_ref, out_ref):
    out_ref[...] = jnp.maximum(_dot(w_ref[...], xt_ref[...]) + b_ref[...], 0.0)


def _lin0_call(xt, gp):
    return pl.pallas_call(
        _lin0_kernel,
        out_shape=jax.ShapeDtypeStruct((HID, N), _f32),
    )(xt, gp['lin0_W'].T, gp['lin0_b'].reshape(HID, 1))


def _mp_kernel(xt_ref, ew_ref, src_ref, dst_ref, out_ref):
    sub_n = jax.lax.broadcasted_iota(jnp.int32, (N, EB_MP), 0).astype(_f32)
    oh_src_t = (src_ref[0] == sub_n).astype(_f32)
    x_src_t = _dot(xt_ref[...], oh_src_t)

    msg_t = jnp.zeros((HID, EB_MP), _f32)
    for i in range(HID):
        msg_t = msg_t + x_src_t[i:i + 1] * ew_ref[i]

    lane_n = jax.lax.broadcasted_iota(jnp.int32, (EB_MP, N), 1).astype(_f32)
    oh_dst = (dst_ref[...] == lane_n).astype(_f32)
    msg_aug = jnp.concatenate([msg_t, jnp.ones((1, EB_MP), _f32)], axis=0)

    @pl.when(pl.program_id(0) == 0)
    def _():
        out_ref[...] = jnp.zeros_like(out_ref)

    out_ref[...] += _dot(msg_aug, oh_dst)


def _mp_call(xt, ew, src_row3, dst_col):
    grid = E // EB_MP
    return pl.pallas_call(
        _mp_kernel,
        grid=(grid,),
        in_specs=[
            pl.BlockSpec((HID, N), lambda i: (0, 0)),
            pl.BlockSpec((HID, HID, EB_MP), lambda i: (0, 0, i)),
            pl.BlockSpec((1, 1, EB_MP), lambda i: (i, 0, 0)),
            pl.BlockSpec((EB_MP, 1), lambda i: (i, 0)),
        ],
        out_specs=pl.BlockSpec((HID + 1, N), lambda i: (0, 0)),
        out_shape=jax.ShapeDtypeStruct((HID + 1, N), _f32),
    )(xt, ew, src_row3, dst_col)


def _upd_kernel(agg_ref, xt_ref, rw_ref, cb_ref, mw_ref, mb_ref, out_ref):
    agg = agg_ref[...]
    cnt = jnp.maximum(agg[HID:HID + 1], 1.0)
    conv = agg[:HID] / cnt + _dot(rw_ref[...], xt_ref[...]) + cb_ref[...]
    m = jnp.maximum(conv, 0.0)
    cat = jnp.concatenate([m, xt_ref[...]], axis=0)
    out_ref[...] = _dot(mw_ref[...], cat) + mb_ref[...]


def _upd_call(agg, xt, gp):
    return pl.pallas_call(
        _upd_kernel,
        out_shape=jax.ShapeDtypeStruct((HID, N), _f32),
    )(agg, xt, gp['root_W'].T, gp['conv_b'].reshape(HID, 1),
      gp['msg_W'].T, gp['msg_b'].reshape(HID, 1))


def _norm_kernel(out3_ref, init_ref, out_ref):
    uf = out3_ref[...] + init_ref[...]
    nrm = jnp.sqrt(jnp.sum(uf * uf, axis=0, keepdims=True))
    out_ref[...] = uf / (nrm + 1e-12)


def _norm_call(out3, init_t):
    return pl.pallas_call(
        _norm_kernel,
        out_shape=jax.ShapeDtypeStruct((HID, N), _f32),
    )(out3, init_t)


def _imap_kernel(nut_b_ref, nur_b_ref, nvt_ref, nvr_ref, sub_col_ref,
                 svb_row_ref, svb_col_ref, sub_row_ref, up_ref, vp_ref):
    nut_b = nut_b_ref[...]
    nvt = nvt_ref[...]
    mask = (sub_col_ref[...] == svb_row_ref[...]).astype(_f32)
    imap_b = _dot(nur_b_ref[...], nvt) * mask

    mask_t = (svb_col_ref[...] == sub_row_ref[0]).astype(_f32)
    imap_bt = _dot(nvr_ref[...], nut_b) * mask_t

    up_ref[...] = _dot(nvt, imap_bt)

    @pl.when(pl.program_id(0) == 0)
    def _():
        vp_ref[...] = jnp.zeros_like(vp_ref)

    vp_ref[...] += _dot(nut_b, imap_b)


def _imap_call(nut, nvt, sub_col, svb_row, svb_col, sub_row3):
    grid = N // RB_IM
    return pl.pallas_call(
        _imap_kernel,
        grid=(grid,),
        in_specs=[
            pl.BlockSpec((HID, RB_IM), lambda i: (0, i)),
            pl.BlockSpec((RB_IM, HID), lambda i: (i, 0)),
            pl.BlockSpec((HID, N), lambda i: (0, 0)),
            pl.BlockSpec((N, HID), lambda i: (0, 0)),
            pl.BlockSpec((RB_IM, 1), lambda i: (i, 0)),
            pl.BlockSpec((1, N), lambda i: (0, 0)),
            pl.BlockSpec((N, 1), lambda i: (0, 0)),
            pl.BlockSpec((1, 1, RB_IM), lambda i: (i, 0, 0)),
        ],
        out_specs=[
            pl.BlockSpec((HID, RB_IM), lambda i: (0, i)),
            pl.BlockSpec((HID, N), lambda i: (0, 0)),
        ],
        out_shape=[
            jax.ShapeDtypeStruct((HID, N), _f32),
            jax.ShapeDtypeStruct((HID, N), _f32),
        ],
    )(nut, nut.T, nvt, nvt.T, sub_col, svb_row, svb_col, sub_row3)


def _s2s_kernel(na_ref, nb_ref, b_col_ref, b_row_ref, wih_ref, whh_ref,
                bb_ref, out_ref):
    d = 2 * HID
    xt = jnp.concatenate([na_ref[...], nb_ref[...]], axis=0)
    oh_bn = (b_row_ref[...] ==
             jax.lax.broadcasted_iota(jnp.int32, (B, N), 0).astype(_f32)
             ).astype(_f32)
    oh_nb = (b_col_ref[...] ==
             jax.lax.broadcasted_iota(jnp.int32, (N, B), 1).astype(_f32)
             ).astype(_f32)

    q_star = jnp.zeros((2 * d, B), _f32)
    h = jnp.zeros((d, B), _f32)
    c = jnp.zeros((d, B), _f32)
    for _ in range(2):
        g = _dot(wih_ref[...], q_star) + _dot(whh_ref[...], h) + bb_ref[...]
        i_ = jax.nn.sigmoid(g[:d])
        f_ = jax.nn.sigmoid(g[d:2 * d])
        gg = jnp.tanh(g[2 * d:3 * d])
        o_ = jax.nn.sigmoid(g[3 * d:])
        c = f_ * c + i_ * gg
        h = o_ * jnp.tanh(c)
        qn = _dot(h, oh_bn)
        e = jnp.sum(xt * qn, axis=0, keepdims=True)
        m2 = jnp.where(oh_bn > 0.5, e, -1e30)
        emax = jnp.max(m2, axis=1, keepdims=True)
        emax = jnp.where(emax > -1e29, emax, 0.0)
        emax_n = jnp.sum(oh_bn * emax, axis=0, keepdims=True)
        ex = jnp.exp(e - emax_n)
        den = jnp.sum(oh_bn * ex, axis=1, keepdims=True)
        den_n = jnp.sum(oh_bn * den, axis=0, keepdims=True)
        a = ex / (den_n + 1e-16)
        r = _dot(a * xt, oh_nb)
        q_star = jnp.concatenate([h, r], axis=0)
    out_ref[...] = q_star


def _s2s_call(nat, nbt, b_col, b_row, sp):
    d = 2 * HID
    return pl.pallas_call(
        _s2s_kernel,
        out_shape=jax.ShapeDtypeStruct((2 * d, B), _f32),
    )(nat, nbt, b_col, b_row, sp['Wih'], sp['Whh'],
      (sp['bih'] + sp['bhh']).reshape(4 * d, 1))


def _pred_kernel(us_ref, vs_ref, w1_ref, b1_ref, w2_ref, b2_ref, w3_ref,
                 b3_ref, out_ref):
    sol = jnp.concatenate([us_ref[...], vs_ref[...]], axis=0)
    h1 = jnp.maximum(_dot(w1_ref[...], sol) + b1_ref[...], 0.0)
    h2 = jnp.maximum(_dot(w2_ref[...], h1) + b2_ref[...], 0.0)
    out_ref[...] = _dot(w3_ref[...], h2) + b3_ref[...]


def _pred_call(us, vs, pp):
    return pl.pallas_call(
        _pred_kernel,
        out_shape=jax.ShapeDtypeStruct((1, B), _f32),
    )(us, vs, pp['W1'].T, pp['b1'].reshape(-1, 1), pp['W2'].T,
      pp['b2'].reshape(-1, 1), pp['W3'].T, pp['b3'].reshape(1, 1))


def _gather_model(xt, ei, eat, gp):
    ew = _ew_call(eat, gp)
    src_row3 = ei[0].astype(_f32).reshape(E // EB_MP, 1, EB_MP)
    dst_col = ei[1].astype(_f32).reshape(E, 1)
    out = _lin0_call(xt, gp)
    for _ in range(3):
        agg = _mp_call(out, ew, src_row3, dst_col)
        out = _upd_call(agg, out, gp)
    return _norm_call(out, xt)


def kernel(solute_x, solvent_x, solute_edge_attr, solvent_edge_attr, params,
           solute_edge_index, solvent_edge_index, solute_batch, solvent_batch):
    nut = _gather_model(solute_x.T, solute_edge_index, solute_edge_attr.T,
                        params['su'])
    nvt = _gather_model(solvent_x.T, solvent_edge_index, solvent_edge_attr.T,
                        params['sv'])

    sub_f = solute_batch.astype(_f32)
    svb_f = solvent_batch.astype(_f32)
    u_prime_t, v_prime_t = _imap_call(
        nut, nvt,
        sub_f.reshape(N, 1), svb_f.reshape(1, N), svb_f.reshape(N, 1),
        sub_f.reshape(N // RB_IM, 1, RB_IM))

    us = _s2s_call(nut, u_prime_t, sub_f.reshape(N, 1), sub_f.reshape(1, N),
                   params['s2s_u'])
    vs = _s2s_call(nvt, v_prime_t, svb_f.reshape(N, 1), svb_f.reshape(1, N),
                   params['s2s_v'])
    return _pred_call(us, vs, params['pred']).reshape(B, 1)


# speedup vs baseline: 3.0458x; 1.0968x over previous
import functools

import jax
import jax.numpy as jnp
from jax.experimental import pallas as pl

HID = 52
N = 4096
E = 16384
B = 256
EB_EW = 512
EB_MP = 512
RB_IM = 512

_f32 = jnp.float32
_HIGHEST = jax.lax.Precision.HIGHEST


def _dot(a, b):
    return jnp.dot(a, b, preferred_element_type=_f32, precision=_HIGHEST)


def _dotd(a, b):
    return jnp.dot(a, b, preferred_element_type=_f32)


def _ew_kernel(eat_ref, w1_ref, b1_ref, w2_ref, b2_ref, out_ref):
    r = jnp.maximum(_dotd(w1_ref[...], eat_ref[...]) + b1_ref[...], 0.0)
    ew_flat = _dotd(w2_ref[...], r) + b2_ref[...]
    out_ref[...] = ew_flat.reshape(HID, HID, EB_EW)


def _ew_call(eat, gp):
    grid = E // EB_EW
    return pl.pallas_call(
        _ew_kernel,
        grid=(grid,),
        in_specs=[
            pl.BlockSpec((10, EB_EW), lambda i: (0, i)),
            pl.BlockSpec((HID, 10), lambda i: (0, 0)),
            pl.BlockSpec((HID, 1), lambda i: (0, 0)),
            pl.BlockSpec((HID * HID, HID), lambda i: (0, 0)),
            pl.BlockSpec((HID * HID, 1), lambda i: (0, 0)),
        ],
        out_specs=pl.BlockSpec((HID, HID, EB_EW), lambda i: (0, 0, i)),
        out_shape=jax.ShapeDtypeStruct((HID, HID, E), _f32),
    )(eat, gp['en1_W'].T, gp['en1_b'].reshape(HID, 1), gp['en2_W'].T,
      gp['en2_b'].reshape(HID * HID, 1))


def _lin0_kernel(xt_ref, w_ref, b_ref, out_ref):
    out_ref[...] = jnp.maximum(_dotd(w_ref[...], xt_ref[...]) + b_ref[...], 0.0)


def _lin0_call(xt, gp):
    return pl.pallas_call(
        _lin0_kernel,
        out_shape=jax.ShapeDtypeStruct((HID, N), _f32),
    )(xt, gp['lin0_W'].T, gp['lin0_b'].reshape(HID, 1))


def _mp_kernel(xt_ref, ew_ref, src_ref, dst_ref, out_ref):
    sub_n = jax.lax.broadcasted_iota(jnp.int32, (N, EB_MP), 0).astype(_f32)
    oh_src_t = (src_ref[0] == sub_n).astype(_f32)
    x_src_t = _dot(xt_ref[...], oh_src_t)

    ew_b = ew_ref[...].astype(jnp.bfloat16).astype(_f32)
    x_b = x_src_t.astype(jnp.bfloat16).astype(_f32)
    msg_t = jnp.sum(ew_b * x_b[:, None, :], axis=0)

    lane_n = jax.lax.broadcasted_iota(jnp.int32, (EB_MP, N), 1).astype(_f32)
    oh_dst = (dst_ref[...] == lane_n).astype(_f32)
    msg_aug = jnp.concatenate([msg_t, jnp.ones((1, EB_MP), _f32)], axis=0)

    @pl.when(pl.program_id(0) == 0)
    def _():
        out_ref[...] = jnp.zeros_like(out_ref)

    out_ref[...] += _dot(msg_aug, oh_dst)


def _mp_call(xt, ew, src_row3, dst_col):
    grid = E // EB_MP
    return pl.pallas_call(
        _mp_kernel,
        grid=(grid,),
        in_specs=[
            pl.BlockSpec((HID, N), lambda i: (0, 0)),
            pl.BlockSpec((HID, HID, EB_MP), lambda i: (0, 0, i)),
            pl.BlockSpec((1, 1, EB_MP), lambda i: (i, 0, 0)),
            pl.BlockSpec((EB_MP, 1), lambda i: (i, 0)),
        ],
        out_specs=pl.BlockSpec((HID + 1, N), lambda i: (0, 0)),
        out_shape=jax.ShapeDtypeStruct((HID + 1, N), _f32),
    )(xt, ew, src_row3, dst_col)


def _upd_kernel(agg_ref, xt_ref, rw_ref, cb_ref, mw_ref, mb_ref, out_ref):
    agg = agg_ref[...]
    cnt = jnp.maximum(agg[HID:HID + 1], 1.0)
    conv = agg[:HID] / cnt + _dotd(rw_ref[...], xt_ref[...]) + cb_ref[...]
    m = jnp.maximum(conv, 0.0)
    cat = jnp.concatenate([m, xt_ref[...]], axis=0)
    out_ref[...] = _dotd(mw_ref[...], cat) + mb_ref[...]


def _upd_call(agg, xt, gp):
    return pl.pallas_call(
        _upd_kernel,
        out_shape=jax.ShapeDtypeStruct((HID, N), _f32),
    )(agg, xt, gp['root_W'].T, gp['conv_b'].reshape(HID, 1),
      gp['msg_W'].T, gp['msg_b'].reshape(HID, 1))


def _norm_kernel(out3_ref, init_ref, out_ref):
    uf = out3_ref[...] + init_ref[...]
    nrm = jnp.sqrt(jnp.sum(uf * uf, axis=0, keepdims=True))
    out_ref[...] = uf / (nrm + 1e-12)


def _norm_call(out3, init_t):
    return pl.pallas_call(
        _norm_kernel,
        out_shape=jax.ShapeDtypeStruct((HID, N), _f32),
    )(out3, init_t)


def _imap_kernel(nut_b_ref, nur_b_ref, nvt_ref, nvr_ref, sub_col_ref,
                 svb_row_ref, svb_col_ref, sub_row_ref, up_ref, vp_ref):
    nut_b = nut_b_ref[...]
    nvt = nvt_ref[...]
    mask = (sub_col_ref[...] == svb_row_ref[...]).astype(_f32)
    imap_b = _dotd(nur_b_ref[...], nvt) * mask

    mask_t = (svb_col_ref[...] == sub_row_ref[0]).astype(_f32)
    imap_bt = _dotd(nvr_ref[...], nut_b) * mask_t

    up_ref[...] = _dotd(nvt, imap_bt)

    @pl.when(pl.program_id(0) == 0)
    def _():
        vp_ref[...] = jnp.zeros_like(vp_ref)

    vp_ref[...] += _dotd(nut_b, imap_b)


def _imap_call(nut, nvt, sub_col, svb_row, svb_col, sub_row3):
    grid = N // RB_IM
    return pl.pallas_call(
        _imap_kernel,
        grid=(grid,),
        in_specs=[
            pl.BlockSpec((HID, RB_IM), lambda i: (0, i)),
            pl.BlockSpec((RB_IM, HID), lambda i: (i, 0)),
            pl.BlockSpec((HID, N), lambda i: (0, 0)),
            pl.BlockSpec((N, HID), lambda i: (0, 0)),
            pl.BlockSpec((RB_IM, 1), lambda i: (i, 0)),
            pl.BlockSpec((1, N), lambda i: (0, 0)),
            pl.BlockSpec((N, 1), lambda i: (0, 0)),
            pl.BlockSpec((1, 1, RB_IM), lambda i: (i, 0, 0)),
        ],
        out_specs=[
            pl.BlockSpec((HID, RB_IM), lambda i: (0, i)),
            pl.BlockSpec((HID, N), lambda i: (0, 0)),
        ],
        out_shape=[
            jax.ShapeDtypeStruct((HID, N), _f32),
            jax.ShapeDtypeStruct((HID, N), _f32),
        ],
    )(nut, nut.T, nvt, nvt.T, sub_col, svb_row, svb_col, sub_row3)


def _s2s_kernel(na_ref, nb_ref, b_col_ref, b_row_ref, wih_ref, whh_ref,
                bb_ref, out_ref):
    d = 2 * HID
    xt = jnp.concatenate([na_ref[...], nb_ref[...]], axis=0)
    oh_bn = (b_row_ref[...] ==
             jax.lax.broadcasted_iota(jnp.int32, (B, N), 0).astype(_f32)
             ).astype(_f32)
    oh_nb = (b_col_ref[...] ==
             jax.lax.broadcasted_iota(jnp.int32, (N, B), 1).astype(_f32)
             ).astype(_f32)

    q_star = jnp.zeros((2 * d, B), _f32)
    h = jnp.zeros((d, B), _f32)
    c = jnp.zeros((d, B), _f32)
    for _ in range(2):
        g = _dotd(wih_ref[...], q_star) + _dotd(whh_ref[...], h) + bb_ref[...]
        i_ = jax.nn.sigmoid(g[:d])
        f_ = jax.nn.sigmoid(g[d:2 * d])
        gg = jnp.tanh(g[2 * d:3 * d])
        o_ = jax.nn.sigmoid(g[3 * d:])
        c = f_ * c + i_ * gg
        h = o_ * jnp.tanh(c)
        qn = _dot(h, oh_bn)
        e = jnp.sum(xt * qn, axis=0, keepdims=True)
        m2 = jnp.where(oh_bn > 0.5, e, -1e30)
        emax = jnp.max(m2, axis=1, keepdims=True)
        emax = jnp.where(emax > -1e29, emax, 0.0)
        emax_n = jnp.sum(oh_bn * emax, axis=0, keepdims=True)
        ex = jnp.exp(e - emax_n)
        den = jnp.sum(oh_bn * ex, axis=1, keepdims=True)
        den_n = jnp.sum(oh_bn * den, axis=0, keepdims=True)
        a = ex / (den_n + 1e-16)
        r = _dot(a * xt, oh_nb)
        q_star = jnp.concatenate([h, r], axis=0)
    out_ref[...] = q_star


def _s2s_call(nat, nbt, b_col, b_row, sp):
    d = 2 * HID
    return pl.pallas_call(
        _s2s_kernel,
        out_shape=jax.ShapeDtypeStruct((2 * d, B), _f32),
    )(nat, nbt, b_col, b_row, sp['Wih'], sp['Whh'],
      (sp['bih'] + sp['bhh']).reshape(4 * d, 1))


def _pred_kernel(us_ref, vs_ref, w1_ref, b1_ref, w2_ref, b2_ref, w3_ref,
                 b3_ref, out_ref):
    sol = jnp.concatenate([us_ref[...], vs_ref[...]], axis=0)
    h1 = jnp.maximum(_dotd(w1_ref[...], sol) + b1_ref[...], 0.0)
    h2 = jnp.maximum(_dotd(w2_ref[...], h1) + b2_ref[...], 0.0)
    out_ref[...] = _dotd(w3_ref[...], h2) + b3_ref[...]


def _pred_call(us, vs, pp):
    return pl.pallas_call(
        _pred_kernel,
        out_shape=jax.ShapeDtypeStruct((1, B), _f32),
    )(us, vs, pp['W1'].T, pp['b1'].reshape(-1, 1), pp['W2'].T,
      pp['b2'].reshape(-1, 1), pp['W3'].T, pp['b3'].reshape(1, 1))


def _gather_model(xt, ei, eat, gp):
    ew = _ew_call(eat, gp)
    src_row3 = ei[0].astype(_f32).reshape(E // EB_MP, 1, EB_MP)
    dst_col = ei[1].astype(_f32).reshape(E, 1)
    out = _lin0_call(xt, gp)
    for _ in range(3):
        agg = _mp_call(out, ew, src_row3, dst_col)
        out = _upd_call(agg, out, gp)
    return _norm_call(out, xt)


def kernel(solute_x, solvent_x, solute_edge_attr, solvent_edge_attr, params,
           solute_edge_index, solvent_edge_index, solute_batch, solvent_batch):
    nut = _gather_model(solute_x.T, solute_edge_index, solute_edge_attr.T,
                        params['su'])
    nvt = _gather_model(solvent_x.T, solvent_edge_index, solvent_edge_attr.T,
                        params['sv'])

    sub_f = solute_batch.astype(_f32)
    svb_f = solvent_batch.astype(_f32)
    u_prime_t, v_prime_t = _imap_call(
        nut, nvt,
        sub_f.reshape(N, 1), svb_f.reshape(1, N), svb_f.reshape(N, 1),
        sub_f.reshape(N // RB_IM, 1, RB_IM))

    us = _s2s_call(nut, u_prime_t, sub_f.reshape(N, 1), sub_f.reshape(1, N),
                   params['s2s_u'])
    vs = _s2s_call(nvt, v_prime_t, svb_f.reshape(N, 1), svb_f.reshape(1, N),
                   params['s2s_v'])
    return _pred_call(us, vs, params['pred']).reshape(B, 1)


# speedup vs baseline: 7.2623x; 2.3844x over previous
import functools

import jax
import jax.numpy as jnp
from jax.experimental import pallas as pl

HID = 52
N = 4096
E = 16384
B = 256
EB_EW = 1024
EB_MP = 1024
RB_IM = 512

_f32 = jnp.float32
_HIGHEST = jax.lax.Precision.HIGHEST


def _dot(a, b):
    return jnp.dot(a, b, preferred_element_type=_f32, precision=_HIGHEST)


def _dotd(a, b):
    return jnp.dot(a, b, preferred_element_type=_f32)


def _ew_kernel(eat_ref, w1_ref, b1_ref, w2_ref, b2_ref, out_ref):
    r = jnp.maximum(_dotd(w1_ref[...], eat_ref[...]) + b1_ref[...], 0.0)
    ew_flat = _dotd(w2_ref[...], r) + b2_ref[...]
    out_ref[...] = ew_flat.reshape(HID, HID, EB_EW).astype(jnp.bfloat16)


def _ew_call(eat, gp):
    grid = E // EB_EW
    return pl.pallas_call(
        _ew_kernel,
        grid=(grid,),
        in_specs=[
            pl.BlockSpec((10, EB_EW), lambda i: (0, i)),
            pl.BlockSpec((HID, 10), lambda i: (0, 0)),
            pl.BlockSpec((HID, 1), lambda i: (0, 0)),
            pl.BlockSpec((HID * HID, HID), lambda i: (0, 0)),
            pl.BlockSpec((HID * HID, 1), lambda i: (0, 0)),
        ],
        out_specs=pl.BlockSpec((HID, HID, EB_EW), lambda i: (0, 0, i)),
        out_shape=jax.ShapeDtypeStruct((HID, HID, E), jnp.bfloat16),
    )(eat, gp['en1_W'].T, gp['en1_b'].reshape(HID, 1), gp['en2_W'].T,
      gp['en2_b'].reshape(HID * HID, 1))


def _lin0_kernel(xt_ref, w_ref, b_ref, out_ref):
    out_ref[...] = jnp.maximum(_dotd(w_ref[...], xt_ref[...]) + b_ref[...], 0.0)


def _lin0_call(xt, gp):
    return pl.pallas_call(
        _lin0_kernel,
        out_shape=jax.ShapeDtypeStruct((HID, N), _f32),
    )(xt, gp['lin0_W'].T, gp['lin0_b'].reshape(HID, 1))


def _mp_kernel(xt_ref, ew_ref, src_ref, dst_ref, out_ref):
    bf16 = jnp.bfloat16
    sub_n = jax.lax.broadcasted_iota(jnp.int32, (N, EB_MP), 0).astype(_f32)
    oh_src_t = (src_ref[0] == sub_n).astype(bf16)
    x_src_t = _dotd(xt_ref[...].astype(bf16), oh_src_t)

    msg_t = jnp.sum(ew_ref[...].astype(_f32) * x_src_t[:, None, :], axis=0)

    lane_n = jax.lax.broadcasted_iota(jnp.int32, (EB_MP, N), 1).astype(_f32)
    oh_dst = (dst_ref[...] == lane_n).astype(bf16)

    m_hi = msg_t.astype(bf16)
    r1 = msg_t - m_hi.astype(_f32)
    m_mid = r1.astype(bf16)
    m_lo = (r1 - m_mid.astype(_f32)).astype(bf16)
    m1_aug = jnp.concatenate([m_hi, jnp.ones((1, EB_MP), bf16)], axis=0)
    acc = _dotd(m1_aug, oh_dst)
    acc2 = _dotd(m_mid, oh_dst) + _dotd(m_lo, oh_dst)
    upd = acc + jnp.concatenate([acc2, jnp.zeros((1, N), _f32)], axis=0)

    @pl.when(pl.program_id(0) == 0)
    def _():
        out_ref[...] = jnp.zeros_like(out_ref)

    out_ref[...] += upd


def _mp_call(xt, ew, src_row3, dst_col):
    grid = E // EB_MP
    return pl.pallas_call(
        _mp_kernel,
        grid=(grid,),
        in_specs=[
            pl.BlockSpec((HID, N), lambda i: (0, 0)),
            pl.BlockSpec((HID, HID, EB_MP), lambda i: (0, 0, i)),
            pl.BlockSpec((1, 1, EB_MP), lambda i: (i, 0, 0)),
            pl.BlockSpec((EB_MP, 1), lambda i: (i, 0)),
        ],
        out_specs=pl.BlockSpec((HID + 1, N), lambda i: (0, 0)),
        out_shape=jax.ShapeDtypeStruct((HID + 1, N), _f32),
    )(xt, ew, src_row3, dst_col)


def _upd_kernel(agg_ref, xt_ref, rw_ref, cb_ref, mw_ref, mb_ref, out_ref):
    agg = agg_ref[...]
    cnt = jnp.maximum(agg[HID:HID + 1], 1.0)
    conv = agg[:HID] / cnt + _dotd(rw_ref[...], xt_ref[...]) + cb_ref[...]
    m = jnp.maximum(conv, 0.0)
    cat = jnp.concatenate([m, xt_ref[...]], axis=0)
    out_ref[...] = _dotd(mw_ref[...], cat) + mb_ref[...]


def _upd_call(agg, xt, gp):
    return pl.pallas_call(
        _upd_kernel,
        out_shape=jax.ShapeDtypeStruct((HID, N), _f32),
    )(agg, xt, gp['root_W'].T, gp['conv_b'].reshape(HID, 1),
      gp['msg_W'].T, gp['msg_b'].reshape(HID, 1))


def _norm_kernel(out3_ref, init_ref, out_ref):
    uf = out3_ref[...] + init_ref[...]
    nrm = jnp.sqrt(jnp.sum(uf * uf, axis=0, keepdims=True))
    out_ref[...] = uf / (nrm + 1e-12)


def _norm_call(out3, init_t):
    return pl.pallas_call(
        _norm_kernel,
        out_shape=jax.ShapeDtypeStruct((HID, N), _f32),
    )(out3, init_t)


def _imap_kernel(nut_b_ref, nur_b_ref, nvt_ref, nvr_ref, sub_col_ref,
                 svb_row_ref, svb_col_ref, sub_row_ref, up_ref, vp_ref):
    nut_b = nut_b_ref[...]
    nvt = nvt_ref[...]
    mask = (sub_col_ref[...] == svb_row_ref[...]).astype(_f32)
    imap_b = _dotd(nur_b_ref[...], nvt) * mask

    mask_t = (svb_col_ref[...] == sub_row_ref[0]).astype(_f32)
    imap_bt = _dotd(nvr_ref[...], nut_b) * mask_t

    up_ref[...] = _dotd(nvt, imap_bt)

    @pl.when(pl.program_id(0) == 0)
    def _():
        vp_ref[...] = jnp.zeros_like(vp_ref)

    vp_ref[...] += _dotd(nut_b, imap_b)


def _imap_call(nut, nvt, sub_col, svb_row, svb_col, sub_row3):
    grid = N // RB_IM
    return pl.pallas_call(
        _imap_kernel,
        grid=(grid,),
        in_specs=[
            pl.BlockSpec((HID, RB_IM), lambda i: (0, i)),
            pl.BlockSpec((RB_IM, HID), lambda i: (i, 0)),
            pl.BlockSpec((HID, N), lambda i: (0, 0)),
            pl.BlockSpec((N, HID), lambda i: (0, 0)),
            pl.BlockSpec((RB_IM, 1), lambda i: (i, 0)),
            pl.BlockSpec((1, N), lambda i: (0, 0)),
            pl.BlockSpec((N, 1), lambda i: (0, 0)),
            pl.BlockSpec((1, 1, RB_IM), lambda i: (i, 0, 0)),
        ],
        out_specs=[
            pl.BlockSpec((HID, RB_IM), lambda i: (0, i)),
            pl.BlockSpec((HID, N), lambda i: (0, 0)),
        ],
        out_shape=[
            jax.ShapeDtypeStruct((HID, N), _f32),
            jax.ShapeDtypeStruct((HID, N), _f32),
        ],
    )(nut, nut.T, nvt, nvt.T, sub_col, svb_row, svb_col, sub_row3)


def _s2s_kernel(na_ref, nb_ref, b_col_ref, b_row_ref, wih_ref, whh_ref,
                bb_ref, out_ref):
    d = 2 * HID
    xt = jnp.concatenate([na_ref[...], nb_ref[...]], axis=0)
    oh_bn = (b_row_ref[...] ==
             jax.lax.broadcasted_iota(jnp.int32, (B, N), 0).astype(_f32)
             ).astype(_f32)
    oh_nb = (b_col_ref[...] ==
             jax.lax.broadcasted_iota(jnp.int32, (N, B), 1).astype(_f32)
             ).astype(_f32)

    q_star = jnp.zeros((2 * d, B), _f32)
    h = jnp.zeros((d, B), _f32)
    c = jnp.zeros((d, B), _f32)
    for _ in range(2):
        g = _dotd(wih_ref[...], q_star) + _dotd(whh_ref[...], h) + bb_ref[...]
        i_ = jax.nn.sigmoid(g[:d])
        f_ = jax.nn.sigmoid(g[d:2 * d])
        gg = jnp.tanh(g[2 * d:3 * d])
        o_ = jax.nn.sigmoid(g[3 * d:])
        c = f_ * c + i_ * gg
        h = o_ * jnp.tanh(c)
        qn = _dot(h, oh_bn)
        e = jnp.sum(xt * qn, axis=0, keepdims=True)
        m2 = jnp.where(oh_bn > 0.5, e, -1e30)
        emax = jnp.max(m2, axis=1, keepdims=True)
        emax = jnp.where(emax > -1e29, emax, 0.0)
        emax_n = jnp.sum(oh_bn * emax, axis=0, keepdims=True)
        ex = jnp.exp(e - emax_n)
        den = jnp.sum(oh_bn * ex, axis=1, keepdims=True)
        den_n = jnp.sum(oh_bn * den, axis=0, keepdims=True)
        a = ex / (den_n + 1e-16)
        r = _dot(a * xt, oh_nb)
        q_star = jnp.concatenate([h, r], axis=0)
    out_ref[...] = q_star


def _s2s_call(nat, nbt, b_col, b_row, sp):
    d = 2 * HID
    return pl.pallas_call(
        _s2s_kernel,
        out_shape=jax.ShapeDtypeStruct((2 * d, B), _f32),
    )(nat, nbt, b_col, b_row, sp['Wih'], sp['Whh'],
      (sp['bih'] + sp['bhh']).reshape(4 * d, 1))


def _pred_kernel(us_ref, vs_ref, w1_ref, b1_ref, w2_ref, b2_ref, w3_ref,
                 b3_ref, out_ref):
    sol = jnp.concatenate([us_ref[...], vs_ref[...]], axis=0)
    h1 = jnp.maximum(_dotd(w1_ref[...], sol) + b1_ref[...], 0.0)
    h2 = jnp.maximum(_dotd(w2_ref[...], h1) + b2_ref[...], 0.0)
    out_ref[...] = _dotd(w3_ref[...], h2) + b3_ref[...]


def _pred_call(us, vs, pp):
    return pl.pallas_call(
        _pred_kernel,
        out_shape=jax.ShapeDtypeStruct((1, B), _f32),
    )(us, vs, pp['W1'].T, pp['b1'].reshape(-1, 1), pp['W2'].T,
      pp['b2'].reshape(-1, 1), pp['W3'].T, pp['b3'].reshape(1, 1))


def _gather_model(xt, ei, eat, gp):
    ew = _ew_call(eat, gp)
    src_row3 = ei[0].astype(_f32).reshape(E // EB_MP, 1, EB_MP)
    dst_col = ei[1].astype(_f32).reshape(E, 1)
    out = _lin0_call(xt, gp)
    for _ in range(3):
        agg = _mp_call(out, ew, src_row3, dst_col)
        out = _upd_call(agg, out, gp)
    return _norm_call(out, xt)


def kernel(solute_x, solvent_x, solute_edge_attr, solvent_edge_attr, params,
           solute_edge_index, solvent_edge_index, solute_batch, solvent_batch):
    nut = _gather_model(solute_x.T, solute_edge_index, solute_edge_attr.T,
                        params['su'])
    nvt = _gather_model(solvent_x.T, solvent_edge_index, solvent_edge_attr.T,
                        params['sv'])

    sub_f = solute_batch.astype(_f32)
    svb_f = solvent_batch.astype(_f32)
    u_prime_t, v_prime_t = _imap_call(
        nut, nvt,
        sub_f.reshape(N, 1), svb_f.reshape(1, N), svb_f.reshape(N, 1),
        sub_f.reshape(N // RB_IM, 1, RB_IM))

    us = _s2s_call(nut, u_prime_t, sub_f.reshape(N, 1), sub_f.reshape(1, N),
                   params['s2s_u'])
    vs = _s2s_call(nvt, v_prime_t, svb_f.reshape(N, 1), svb_f.reshape(1, N),
                   params['s2s_v'])
    return _pred_call(us, vs, params['pred']).reshape(B, 1)
